# baseline (speedup 1.0000x reference)
import functools

import jax
import jax.numpy as jnp
from jax import lax
from jax.experimental import pallas as pl
from jax.experimental.pallas import tpu as pltpu
from jax.experimental.pallas import tpu_sc as plsc

_D = 128
_VOCAB = 1000
_BASE = 10000.0
_PITCH_SIZE = 128
_BAR_ID = 128
_POS_START, _POS_SIZE = 129, 32
_B, _T = 4096, 200
_EXT = 1208

_NC, _NS = 2, 16
_NW = _NC * _NS
_CHUNK = 128
_IDX_ROWS = _B * _T // _CHUNK
_ROWS_PER_TILE = _IDX_ROWS // _NW
_GRP = 8
_NGRP = _ROWS_PER_TILE // _GRP
_TBL_WORDS = _EXT * (_D // 2)


def _table_body(tw_ref, pb_ref, qb_ref, bb_ref, out_ref):
    r = lax.broadcasted_iota(jnp.int32, (_EXT, _D), 0)
    d = lax.broadcasted_iota(jnp.int32, (_EXT, _D), 1)
    k = (d // 2).astype(jnp.float32)
    f = jnp.exp(k * jnp.float32(-2.0 / _D) * jnp.log(jnp.float32(_BASE)))
    is_pitch = r < _PITCH_SIZE
    is_pos = (r >= _POS_START) & (r < _POS_START + _POS_SIZE)
    n = jnp.where(is_pitch, r,
                  jnp.where(is_pos, r - _POS_START, r - _VOCAB)).astype(jnp.float32)
    ang = n * f
    even = (d % 2) == 0
    bias = jnp.where(is_pitch, pb_ref[...],
                     jnp.where(is_pos, qb_ref[...], bb_ref[...]))
    val = jnp.where(even, jnp.sin(ang), jnp.cos(ang)) + bias
    is_fme = is_pitch | is_pos
    out_ref[0:_VOCAB, :] = jnp.where(is_fme[0:_VOCAB, :], val[0:_VOCAB, :],
                                     tw_ref[...])
    out_ref[_VOCAB:_EXT, :] = val[_VOCAB:_EXT, :]


def _build_table(token_weight, pitch_bias, pos_bias, bar_bias):
    return pl.pallas_call(
        _table_body,
        out_shape=jax.ShapeDtypeStruct((_EXT, _D), jnp.float32),
    )(token_weight, pitch_bias, pos_bias, bar_bias)


def _eff_body(idx_ref, out_ref):
    x = idx_ref[...]
    bar = x == _BAR_ID
    barf = bar.astype(jnp.float32)
    ti = lax.broadcasted_iota(jnp.int32, (_T, _T), 0)
    tj = lax.broadcasted_iota(jnp.int32, (_T, _T), 1)
    tril = (ti <= tj).astype(jnp.float32)
    csum = lax.dot(barf, tril, preferred_element_type=jnp.float32)
    bar_val = jnp.maximum(csum.astype(jnp.int32) - 1, 0)
    out_ref[...] = jnp.where(bar, _VOCAB + bar_val, jnp.clip(x, 0, _VOCAB - 1))


def _eff_idx(idx):
    blk = 256
    return pl.pallas_call(
        _eff_body,
        grid=(_B // blk,),
        in_specs=[pl.BlockSpec((blk, _T), lambda i: (i, 0))],
        out_specs=pl.BlockSpec((blk, _T), lambda i: (i, 0)),
        out_shape=jax.ShapeDtypeStruct((_B, _T), jnp.int32),
    )(idx)


def _sc_gather(tbl_i32, idx2d):
    mesh = plsc.VectorSubcoreMesh(core_axis_name="c", subcore_axis_name="s")

    @functools.partial(
        pl.kernel,
        out_type=jax.ShapeDtypeStruct((_B * _T * _D,), jnp.float32),
        mesh=mesh,
        compiler_params=pltpu.CompilerParams(needs_layout_passes=False),
        scratch_types=[
            pltpu.VMEM((_TBL_WORDS,), jnp.int32),
            pltpu.VMEM((_CHUNK * _D,), jnp.float32),
            pltpu.VMEM((_CHUNK * _D,), jnp.float32),
            pltpu.VMEM((2, _GRP, _CHUNK), jnp.int32),
            pltpu.SemaphoreType.DMA,
            pltpu.SemaphoreType.DMA,
            pltpu.SemaphoreType.DMA,
            pltpu.SemaphoreType.DMA,
        ],
    )
    def k(tbl_hbm, idx_hbm, out_hbm, tblv, stage0, stage1, idxv, si0, si1,
          ss0, ss1):
        stages = (stage0, stage1)
        wid = lax.axis_index("s") * _NC + lax.axis_index("c")
        row_base = wid * _ROWS_PER_TILE
        tok_base = row_base * _CHUNK
        sis = (si0, si1)
        sss = (ss0, ss1)

        pltpu.sync_copy(tbl_hbm, tblv)

        def idx_copy(g, buf):
            return pltpu.make_async_copy(
                idx_hbm.at[pl.ds(row_base + g * _GRP, _GRP)], idxv.at[buf],
                sis[buf])

        def scat_copy(c, p):
            return pltpu.make_async_copy(
                stages[p],
                out_hbm.at[pl.ds((tok_base + c * _CHUNK) * _D, _CHUNK * _D)],
                sss[p])

        def chunk(c, j, buf):
            p = j % 2

            @pl.when(c >= 2)
            def _():
                scat_copy(c - 2, p).wait()

            for kk in range(_CHUNK // 16):
                e = idxv[buf, j, pl.ds(kk * 16, 16)]
                ebase = e * (_D // 2)
                ti = (lax.iota(jnp.int32, 16) + kk * 16) * _D

                @plsc.parallel_loop(0, _D // 2, unroll=4)
                def _(d2):
                    w = plsc.load_gather(tblv, [ebase + d2])
                    ab = plsc.bitcast(w, jnp.bfloat16)
                    lo, hi = plsc.unpack(
                        ab, format=plsc.PackFormat.INTERLEAVED,
                        preferred_element_type=jnp.float32)
                    plsc.store_scatter(stages[p], [ti + 2 * d2], lo)
                    plsc.store_scatter(stages[p], [ti + 2 * d2 + 1], hi)

            scat_copy(c, p).start()

        idx_copy(0, 0).start()

        def body(i, carry):
            ga = 2 * i
            idx_copy(ga, 0).wait()

            @pl.when(ga + 1 < _NGRP)
            def _():
                idx_copy(ga + 1, 1).start()

            for j in range(_GRP):
                chunk(ga * _GRP + j, j, 0)

            @pl.when(ga + 2 < _NGRP)
            def _():
                idx_copy(ga + 2, 0).start()

            @pl.when(ga + 1 < _NGRP)
            def _():
                idx_copy(ga + 1, 1).wait()
                for j in range(_GRP):
                    chunk((ga + 1) * _GRP + j, j, 1)

            return carry

        lax.fori_loop(0, (_NGRP + 1) // 2, body, 0)

        scat_copy(_ROWS_PER_TILE - 2, 0).wait()
        scat_copy(_ROWS_PER_TILE - 1, 1).wait()

    return k(tbl_i32, idx2d)


def kernel(idx, token_weight, pitch_bias, pos_bias, bar_bias):
    idx = idx.astype(jnp.int32)
    table = _build_table(token_weight,
                         pitch_bias.reshape(1, _D),
                         pos_bias.reshape(1, _D),
                         bar_bias.reshape(1, _D))
    tbl_bf16 = table.astype(jnp.bfloat16)
    tbl_i32 = lax.bitcast_convert_type(
        tbl_bf16.reshape(_EXT, _D // 2, 2), jnp.int32).reshape(_TBL_WORDS)
    eff = _eff_idx(idx)
    idx2d = eff.reshape(_IDX_ROWS, _CHUNK)
    out = _sc_gather(tbl_i32, idx2d)
    return out.reshape(_B, _T, _D)


# speedup vs baseline: 2.7885x; 2.7885x over previous
import functools

import jax
import jax.numpy as jnp
from jax import lax
from jax.experimental import pallas as pl
from jax.experimental.pallas import tpu as pltpu
from jax.experimental.pallas import tpu_sc as plsc

_D = 128
_VOCAB = 1000
_BASE = 10000.0
_PITCH_SIZE = 128
_BAR_ID = 128
_POS_START, _POS_SIZE = 129, 32
_B, _T = 4096, 200
_EXT = 1208

_NC, _NS = 2, 16
_NW = _NC * _NS
_CHUNK = 128
_IDX_ROWS = _B * _T // _CHUNK
_ROWS_PER_TILE = _IDX_ROWS // _NW
_BLK = 512


def _prep_body(idx_ref, tw_ref, pb_ref, qb_ref, bb_ref, tbl_ref, eff_ref):
    x = idx_ref[...]
    bar = x == _BAR_ID
    barf = bar.astype(jnp.float32)
    ti = lax.broadcasted_iota(jnp.int32, (_T, _T), 0)
    tj = lax.broadcasted_iota(jnp.int32, (_T, _T), 1)
    tril = (ti <= tj).astype(jnp.float32)
    csum = lax.dot(barf, tril, preferred_element_type=jnp.float32)
    bar_val = jnp.maximum(csum.astype(jnp.int32) - 1, 0)
    eff_ref[...] = jnp.where(bar, _VOCAB + bar_val, jnp.clip(x, 0, _VOCAB - 1))

    @pl.when(pl.program_id(0) == 0)
    def _():
        r = lax.broadcasted_iota(jnp.int32, (_EXT, _D), 0)
        d = lax.broadcasted_iota(jnp.int32, (_EXT, _D), 1)
        k = (d // 2).astype(jnp.float32)
        f = jnp.exp(k * jnp.float32(-2.0 / _D) * jnp.log(jnp.float32(_BASE)))
        is_pitch = r < _PITCH_SIZE
        is_pos = (r >= _POS_START) & (r < _POS_START + _POS_SIZE)
        n = jnp.where(is_pitch, r,
                      jnp.where(is_pos, r - _POS_START,
                                r - _VOCAB)).astype(jnp.float32)
        ang = n * f
        even = (d % 2) == 0
        bias = jnp.where(is_pitch, pb_ref[...],
                         jnp.where(is_pos, qb_ref[...], bb_ref[...]))
        val = jnp.where(even, jnp.sin(ang), jnp.cos(ang)) + bias
        is_fme = is_pitch | is_pos
        tbl_ref[0:_VOCAB, :] = jnp.where(is_fme[0:_VOCAB, :], val[0:_VOCAB, :],
                                         tw_ref[...])
        tbl_ref[_VOCAB:_EXT, :] = val[_VOCAB:_EXT, :]


def _prep(idx, token_weight, pitch_bias, pos_bias, bar_bias):
    zero = lambda i: (0, 0)
    return pl.pallas_call(
        _prep_body,
        grid=(_B // _BLK,),
        in_specs=[
            pl.BlockSpec((_BLK, _T), lambda i: (i, 0)),
            pl.BlockSpec((_VOCAB, _D), zero),
            pl.BlockSpec((1, _D), zero),
            pl.BlockSpec((1, _D), zero),
            pl.BlockSpec((1, _D), zero),
        ],
        out_specs=[
            pl.BlockSpec((_EXT, _D), zero),
            pl.BlockSpec((_BLK, _T), lambda i: (i, 0)),
        ],
        out_shape=[
            jax.ShapeDtypeStruct((_EXT, _D), jnp.float32),
            jax.ShapeDtypeStruct((_B, _T), jnp.int32),
        ],
    )(idx, token_weight, pitch_bias, pos_bias, bar_bias)


_G = 3
_NGRP = -(-_ROWS_PER_TILE // _G)


def _sc_gather(table, idx2d):
    mesh = plsc.VectorSubcoreMesh(core_axis_name="c", subcore_axis_name="s")

    @functools.partial(
        pl.kernel,
        out_type=jax.ShapeDtypeStruct((_B * _T, _D), jnp.float32),
        mesh=mesh,
        scratch_types=[
            pltpu.VMEM((_ROWS_PER_TILE, _CHUNK), jnp.int32),
            pltpu.VMEM((2 * _G, _CHUNK, _D), jnp.float32),
            pltpu.SemaphoreType.DMA,
            pltpu.SemaphoreType.DMA,
            pltpu.SemaphoreType.DMA,
            pltpu.SemaphoreType.DMA,
        ],
    )
    def k(table_hbm, idx_hbm, out_hbm, idx_v, rows_v, sg0, sg1, ss0, ss1):
        wid = lax.axis_index("s") * _NC + lax.axis_index("c")
        row_base = wid * _ROWS_PER_TILE
        tok_base = row_base * _CHUNK
        pltpu.sync_copy(idx_hbm.at[pl.ds(row_base, _ROWS_PER_TILE)], idx_v)

        def gathers(g, half, sem, start):
            for j in range(_G):
                c = _G * g + j

                @pl.when(c < _ROWS_PER_TILE)
                def _():
                    cp = pltpu.make_async_copy(
                        table_hbm.at[idx_v.at[c]], rows_v.at[half * _G + j],
                        sem)
                    cp.start() if start else cp.wait()

        def scatters(g, half, sem, start):
            for j in range(_G):
                c = _G * g + j

                @pl.when(c < _ROWS_PER_TILE)
                def _():
                    cp = pltpu.make_async_copy(
                        rows_v.at[half * _G + j],
                        out_hbm.at[pl.ds(tok_base + c * _CHUNK, _CHUNK)], sem)
                    cp.start() if start else cp.wait()

        gathers(0, 0, sg0, True)

        def body(i, carry):
            ga = 2 * i
            gb = 2 * i + 1
            gathers(ga, 0, sg0, False)
            scatters(ga, 0, ss0, True)

            @pl.when(i >= 1)
            def _():
                scatters(ga - 1, 1, ss1, False)

            @pl.when(gb < _NGRP)
            def _():
                gathers(gb, 1, sg1, True)
                gathers(gb, 1, sg1, False)
                scatters(gb, 1, ss1, True)

            scatters(ga, 0, ss0, False)

            @pl.when(ga + 2 < _NGRP)
            def _():
                gathers(ga + 2, 0, sg0, True)

            return carry

        lax.fori_loop(0, (_NGRP + 1) // 2, body, 0)

    return k(table, idx2d)


def kernel(idx, token_weight, pitch_bias, pos_bias, bar_bias):
    idx = idx.astype(jnp.int32)
    table, eff = _prep(idx, token_weight,
                       pitch_bias.reshape(1, _D),
                       pos_bias.reshape(1, _D),
                       bar_bias.reshape(1, _D))
    idx2d = eff.reshape(_IDX_ROWS, _CHUNK)
    out = _sc_gather(table, idx2d)
    return out.reshape(_B, _T, _D)


# speedup vs baseline: 3.0158x; 1.0815x over previous
import functools

import jax
import jax.numpy as jnp
from jax import lax
from jax.experimental import pallas as pl
from jax.experimental.pallas import tpu as pltpu
from jax.experimental.pallas import tpu_sc as plsc

_D = 128
_VOCAB = 1000
_BASE = 10000.0
_PITCH_SIZE = 128
_BAR_ID = 128
_POS_START, _POS_SIZE = 129, 32
_B, _T = 4096, 200
_EXT = 1208

_NC, _NS = 2, 16
_NW = _NC * _NS
_CHUNK = 128
_IDX_ROWS = _B * _T // _CHUNK
_ROWS_PER_TILE = _IDX_ROWS // _NW
_BLK = 512


def _table_body(tw_ref, pb_ref, qb_ref, bb_ref, out_ref):
    r = lax.broadcasted_iota(jnp.int32, (_EXT, _D), 0)
    d = lax.broadcasted_iota(jnp.int32, (_EXT, _D), 1)
    k = (d // 2).astype(jnp.float32)
    f = jnp.exp(k * jnp.float32(-2.0 / _D) * jnp.log(jnp.float32(_BASE)))
    is_pitch = r < _PITCH_SIZE
    is_pos = (r >= _POS_START) & (r < _POS_START + _POS_SIZE)
    n = jnp.where(is_pitch, r,
                  jnp.where(is_pos, r - _POS_START, r - _VOCAB)).astype(jnp.float32)
    ang = n * f
    even = (d % 2) == 0
    bias = jnp.where(is_pitch, pb_ref[...],
                     jnp.where(is_pos, qb_ref[...], bb_ref[...]))
    val = jnp.where(even, jnp.sin(ang), jnp.cos(ang)) + bias
    is_fme = is_pitch | is_pos
    out_ref[0:_VOCAB, :] = jnp.where(is_fme[0:_VOCAB, :], val[0:_VOCAB, :],
                                     tw_ref[...])
    out_ref[_VOCAB:_EXT, :] = val[_VOCAB:_EXT, :]


def _build_table(token_weight, pitch_bias, pos_bias, bar_bias):
    return pl.pallas_call(
        _table_body,
        out_shape=jax.ShapeDtypeStruct((_EXT, _D), jnp.float32),
    )(token_weight, pitch_bias, pos_bias, bar_bias)


def _eff_body(idx_ref, out_ref):
    x = idx_ref[...]
    bar = x == _BAR_ID
    barf = bar.astype(jnp.float32)
    ti = lax.broadcasted_iota(jnp.int32, (_T, _T), 0)
    tj = lax.broadcasted_iota(jnp.int32, (_T, _T), 1)
    tril = (ti <= tj).astype(jnp.float32)
    csum = lax.dot(barf, tril, preferred_element_type=jnp.float32)
    bar_val = jnp.maximum(csum.astype(jnp.int32) - 1, 0)
    out_ref[...] = jnp.where(bar, _VOCAB + bar_val, jnp.clip(x, 0, _VOCAB - 1))


def _eff_idx(idx):
    blk = 256
    return pl.pallas_call(
        _eff_body,
        grid=(_B // blk,),
        in_specs=[pl.BlockSpec((blk, _T), lambda i: (i, 0))],
        out_specs=pl.BlockSpec((blk, _T), lambda i: (i, 0)),
        out_shape=jax.ShapeDtypeStruct((_B, _T), jnp.int32),
    )(idx)


_G = 3
_NGRP = -(-_ROWS_PER_TILE // _G)


def _sc_gather(table, idx2d):
    mesh = plsc.VectorSubcoreMesh(core_axis_name="c", subcore_axis_name="s")

    @functools.partial(
        pl.kernel,
        out_type=jax.ShapeDtypeStruct((_B * _T, _D), jnp.float32),
        mesh=mesh,
        scratch_types=[
            pltpu.VMEM((_ROWS_PER_TILE, _CHUNK), jnp.int32),
            pltpu.VMEM((2 * _G, _CHUNK, _D), jnp.float32),
            pltpu.SemaphoreType.DMA,
            pltpu.SemaphoreType.DMA,
            pltpu.SemaphoreType.DMA,
            pltpu.SemaphoreType.DMA,
        ],
    )
    def k(table_hbm, idx_hbm, out_hbm, idx_v, rows_v, sg0, sg1, ss0, ss1):
        wid = lax.axis_index("s") * _NC + lax.axis_index("c")
        row_base = wid * _ROWS_PER_TILE
        tok_base = row_base * _CHUNK
        pltpu.sync_copy(idx_hbm.at[pl.ds(row_base, _ROWS_PER_TILE)], idx_v)

        def gathers(g, half, sem, start):
            for j in range(_G):
                c = _G * g + j

                @pl.when(c < _ROWS_PER_TILE)
                def _():
                    cp = pltpu.make_async_copy(
                        table_hbm.at[idx_v.at[c]], rows_v.at[half * _G + j],
                        sem)
                    cp.start() if start else cp.wait()

        def scatters(g, half, sem, start):
            for j in range(_G):
                c = _G * g + j

                @pl.when(c < _ROWS_PER_TILE)
                def _():
                    cp = pltpu.make_async_copy(
                        rows_v.at[half * _G + j],
                        out_hbm.at[pl.ds(tok_base + c * _CHUNK, _CHUNK)], sem)
                    cp.start() if start else cp.wait()

        gathers(0, 0, sg0, True)

        def body(i, carry):
            ga = 2 * i
            gb = 2 * i + 1
            gathers(ga, 0, sg0, False)
            scatters(ga, 0, ss0, True)

            @pl.when(i >= 1)
            def _():
                scatters(ga - 1, 1, ss1, False)

            @pl.when(gb < _NGRP)
            def _():
                gathers(gb, 1, sg1, True)
                gathers(gb, 1, sg1, False)
                scatters(gb, 1, ss1, True)

            scatters(ga, 0, ss0, False)

            @pl.when(ga + 2 < _NGRP)
            def _():
                gathers(ga + 2, 0, sg0, True)

            return carry

        lax.fori_loop(0, (_NGRP + 1) // 2, body, 0)

    return k(table, idx2d)


def kernel(idx, token_weight, pitch_bias, pos_bias, bar_bias):
    idx = idx.astype(jnp.int32)
    table = _build_table(token_weight,
                         pitch_bias.reshape(1, _D),
                         pos_bias.reshape(1, _D),
                         bar_bias.reshape(1, _D))
    eff = _eff_idx(idx)
    idx2d = eff.reshape(_IDX_ROWS, _CHUNK)
    out = _sc_gather(table, idx2d)
    return out.reshape(_B, _T, _D)


# speedup vs baseline: 3.1305x; 1.0380x over previous
import functools

import jax
import jax.numpy as jnp
from jax import lax
from jax.experimental import pallas as pl
from jax.experimental.pallas import tpu as pltpu
from jax.experimental.pallas import tpu_sc as plsc

_D = 128
_VOCAB = 1000
_BASE = 10000.0
_PITCH_SIZE = 128
_BAR_ID = 128
_POS_START, _POS_SIZE = 129, 32
_B, _T = 4096, 200
_EXT = 1208

_NC, _NS = 2, 16
_NW = _NC * _NS
_CHUNK = 128
_IDX_ROWS = _B * _T // _CHUNK
_S_ROWS = 1024
_ROWS_PER_TILE = (_IDX_ROWS - _S_ROWS) // _NW


def _table_body(tw_ref, pb_ref, qb_ref, bb_ref, out_ref):
    r = lax.broadcasted_iota(jnp.int32, (_EXT, _D), 0)
    d = lax.broadcasted_iota(jnp.int32, (_EXT, _D), 1)
    k = (d // 2).astype(jnp.float32)
    f = jnp.exp(k * jnp.float32(-2.0 / _D) * jnp.log(jnp.float32(_BASE)))
    is_pitch = r < _PITCH_SIZE
    is_pos = (r >= _POS_START) & (r < _POS_START + _POS_SIZE)
    n = jnp.where(is_pitch, r,
                  jnp.where(is_pos, r - _POS_START, r - _VOCAB)).astype(jnp.float32)
    ang = n * f
    even = (d % 2) == 0
    bias = jnp.where(is_pitch, pb_ref[...],
                     jnp.where(is_pos, qb_ref[...], bb_ref[...]))
    val = jnp.where(even, jnp.sin(ang), jnp.cos(ang)) + bias
    is_fme = is_pitch | is_pos
    out_ref[0:_VOCAB, :] = jnp.where(is_fme[0:_VOCAB, :], val[0:_VOCAB, :],
                                     tw_ref[...])
    out_ref[_VOCAB:_EXT, :] = val[_VOCAB:_EXT, :]


def _build_table(token_weight, pitch_bias, pos_bias, bar_bias):
    return pl.pallas_call(
        _table_body,
        out_shape=jax.ShapeDtypeStruct((_EXT, _D), jnp.float32),
    )(token_weight, pitch_bias, pos_bias, bar_bias)


def _eff_body(idx_ref, out_ref):
    x = idx_ref[...]
    bar = x == _BAR_ID
    barf = bar.astype(jnp.float32)
    ti = lax.broadcasted_iota(jnp.int32, (_T, _T), 0)
    tj = lax.broadcasted_iota(jnp.int32, (_T, _T), 1)
    tril = (ti <= tj).astype(jnp.float32)
    csum = lax.dot(barf, tril, preferred_element_type=jnp.float32)
    bar_val = jnp.maximum(csum.astype(jnp.int32) - 1, 0)
    out_ref[...] = jnp.where(bar, _VOCAB + bar_val, jnp.clip(x, 0, _VOCAB - 1))


def _eff_idx(idx):
    blk = 256
    return pl.pallas_call(
        _eff_body,
        grid=(_B // blk,),
        in_specs=[pl.BlockSpec((blk, _T), lambda i: (i, 0))],
        out_specs=pl.BlockSpec((blk, _T), lambda i: (i, 0)),
        out_shape=jax.ShapeDtypeStruct((_B, _T), jnp.int32),
    )(idx)


_G = 3
_NGRP = -(-_ROWS_PER_TILE // _G)


def _sc_gather(table, idx2d):
    mesh = plsc.VectorSubcoreMesh(core_axis_name="c", subcore_axis_name="s")

    @functools.partial(
        pl.kernel,
        out_type=jax.ShapeDtypeStruct((_B * _T, _D), jnp.float32),
        mesh=mesh,
        scratch_types=[
            pltpu.VMEM((_ROWS_PER_TILE, _CHUNK), jnp.int32),
            pltpu.VMEM((2 * _G, _CHUNK, _D), jnp.float32),
            pltpu.SemaphoreType.DMA,
            pltpu.SemaphoreType.DMA,
            pltpu.SemaphoreType.DMA,
            pltpu.SemaphoreType.DMA,
        ],
    )
    def k(table_hbm, idx_hbm, out_hbm, idx_v, rows_v, sg0, sg1, ss0, ss1):
        wid = lax.axis_index("s") * _NC + lax.axis_index("c")
        row_base = _S_ROWS + wid * _ROWS_PER_TILE
        tok_base = row_base * _CHUNK
        pltpu.sync_copy(idx_hbm.at[pl.ds(row_base, _ROWS_PER_TILE)], idx_v)

        def gathers(g, half, sem, start):
            for j in range(_G):
                c = _G * g + j

                @pl.when(c < _ROWS_PER_TILE)
                def _():
                    cp = pltpu.make_async_copy(
                        table_hbm.at[idx_v.at[c]], rows_v.at[half * _G + j],
                        sem)
                    cp.start() if start else cp.wait()

        def scatters(g, half, sem, start):
            for j in range(_G):
                c = _G * g + j

                @pl.when(c < _ROWS_PER_TILE)
                def _():
                    cp = pltpu.make_async_copy(
                        rows_v.at[half * _G + j],
                        out_hbm.at[pl.ds(tok_base + c * _CHUNK, _CHUNK)], sem)
                    cp.start() if start else cp.wait()

        gathers(0, 0, sg0, True)

        def body(i, carry):
            ga = 2 * i
            gb = 2 * i + 1
            gathers(ga, 0, sg0, False)
            scatters(ga, 0, ss0, True)

            @pl.when(i >= 1)
            def _():
                scatters(ga - 1, 1, ss1, False)

            @pl.when(gb < _NGRP)
            def _():
                gathers(gb, 1, sg1, True)
                gathers(gb, 1, sg1, False)
                scatters(gb, 1, ss1, True)

            scatters(ga, 0, ss0, False)

            @pl.when(ga + 2 < _NGRP)
            def _():
                gathers(ga + 2, 0, sg0, True)

            return carry

        lax.fori_loop(0, (_NGRP + 1) // 2, body, 0)
        if _NGRP % 2 == 0:
            scatters(_NGRP - 1, 1, ss1, False)

    return k(table, idx2d)


def _bf16_body(tbl_ref, out_ref):
    out_ref[...] = tbl_ref[...].astype(jnp.bfloat16)


def _to_bf16(table):
    return pl.pallas_call(
        _bf16_body,
        out_shape=jax.ShapeDtypeStruct((_EXT, _D), jnp.bfloat16),
    )(table)


_TC_SUB = 8


def _tc_body(eff_ref, tbl_ref, out_ref):
    tbl = tbl_ref[...]
    v = lax.broadcasted_iota(jnp.int32, (_EXT, _CHUNK), 0)
    for r in range(_TC_SUB):
        oh = (v == eff_ref[r:r + 1, :]).astype(jnp.bfloat16)
        out_ref[r * _CHUNK:(r + 1) * _CHUNK, :] = lax.dot_general(
            oh, tbl, (((0,), (0,)), ((), ())),
            preferred_element_type=jnp.float32)


def _tc_gather(eff2d, tbl16):
    return pl.pallas_call(
        _tc_body,
        grid=(_S_ROWS // _TC_SUB,),
        in_specs=[
            pl.BlockSpec((_TC_SUB, _CHUNK), lambda i: (i, 0)),
            pl.BlockSpec((_EXT, _D), lambda i: (0, 0)),
        ],
        out_specs=pl.BlockSpec((_TC_SUB * _CHUNK, _D), lambda i: (i, 0)),
        out_shape=jax.ShapeDtypeStruct((_S_ROWS * _CHUNK, _D), jnp.float32),
    )(eff2d, tbl16)


def kernel(idx, token_weight, pitch_bias, pos_bias, bar_bias):
    idx = idx.astype(jnp.int32)
    table = _build_table(token_weight,
                         pitch_bias.reshape(1, _D),
                         pos_bias.reshape(1, _D),
                         bar_bias.reshape(1, _D))
    eff = _eff_idx(idx)
    idx2d = eff.reshape(_IDX_ROWS, _CHUNK)
    tbl16 = _to_bf16(table)
    sc_out = _sc_gather(table, idx2d)
    tc_out = _tc_gather(idx2d, tbl16)
    out = lax.dynamic_update_slice(sc_out, tc_out, (0, 0))
    return out.reshape(_B, _T, _D)


# speedup vs baseline: 3.1384x; 1.0025x over previous
import functools

import jax
import jax.numpy as jnp
from jax import lax
from jax.experimental import pallas as pl
from jax.experimental.pallas import tpu as pltpu
from jax.experimental.pallas import tpu_sc as plsc

_D = 128
_VOCAB = 1000
_BASE = 10000.0
_PITCH_SIZE = 128
_BAR_ID = 128
_POS_START, _POS_SIZE = 129, 32
_B, _T = 4096, 200
_EXT = 1208

_NC, _NS = 2, 16
_NW = _NC * _NS
_CHUNK = 128
_IDX_ROWS = _B * _T // _CHUNK
_S_ROWS = 1280
_ROWS_PER_TILE = (_IDX_ROWS - _S_ROWS) // _NW


def _table_body(tw_ref, pb_ref, qb_ref, bb_ref, out_ref):
    r = lax.broadcasted_iota(jnp.int32, (_EXT, _D), 0)
    d = lax.broadcasted_iota(jnp.int32, (_EXT, _D), 1)
    k = (d // 2).astype(jnp.float32)
    f = jnp.exp(k * jnp.float32(-2.0 / _D) * jnp.log(jnp.float32(_BASE)))
    is_pitch = r < _PITCH_SIZE
    is_pos = (r >= _POS_START) & (r < _POS_START + _POS_SIZE)
    n = jnp.where(is_pitch, r,
                  jnp.where(is_pos, r - _POS_START, r - _VOCAB)).astype(jnp.float32)
    ang = n * f
    even = (d % 2) == 0
    bias = jnp.where(is_pitch, pb_ref[...],
                     jnp.where(is_pos, qb_ref[...], bb_ref[...]))
    val = jnp.where(even, jnp.sin(ang), jnp.cos(ang)) + bias
    is_fme = is_pitch | is_pos
    out_ref[0:_VOCAB, :] = jnp.where(is_fme[0:_VOCAB, :], val[0:_VOCAB, :],
                                     tw_ref[...])
    out_ref[_VOCAB:_EXT, :] = val[_VOCAB:_EXT, :]


def _build_table(token_weight, pitch_bias, pos_bias, bar_bias):
    return pl.pallas_call(
        _table_body,
        out_shape=jax.ShapeDtypeStruct((_EXT, _D), jnp.float32),
    )(token_weight, pitch_bias, pos_bias, bar_bias)


def _eff_body(idx_ref, out_ref):
    x = idx_ref[...]
    bar = x == _BAR_ID
    barf = bar.astype(jnp.float32)
    ti = lax.broadcasted_iota(jnp.int32, (_T, _T), 0)
    tj = lax.broadcasted_iota(jnp.int32, (_T, _T), 1)
    tril = (ti <= tj).astype(jnp.float32)
    csum = lax.dot(barf, tril, preferred_element_type=jnp.float32)
    bar_val = jnp.maximum(csum.astype(jnp.int32) - 1, 0)
    out_ref[...] = jnp.where(bar, _VOCAB + bar_val, jnp.clip(x, 0, _VOCAB - 1))


def _eff_idx(idx):
    blk = 256
    return pl.pallas_call(
        _eff_body,
        grid=(_B // blk,),
        in_specs=[pl.BlockSpec((blk, _T), lambda i: (i, 0))],
        out_specs=pl.BlockSpec((blk, _T), lambda i: (i, 0)),
        out_shape=jax.ShapeDtypeStruct((_B, _T), jnp.int32),
    )(idx)


_G = 3
_NGRP = -(-_ROWS_PER_TILE // _G)


def _sc_gather(table, idx2d):
    mesh = plsc.VectorSubcoreMesh(core_axis_name="c", subcore_axis_name="s")

    @functools.partial(
        pl.kernel,
        out_type=jax.ShapeDtypeStruct((_B * _T, _D), jnp.float32),
        mesh=mesh,
        scratch_types=[
            pltpu.VMEM((_ROWS_PER_TILE, _CHUNK), jnp.int32),
            pltpu.VMEM((2 * _G, _CHUNK, _D), jnp.float32),
            pltpu.SemaphoreType.DMA,
            pltpu.SemaphoreType.DMA,
            pltpu.SemaphoreType.DMA,
            pltpu.SemaphoreType.DMA,
        ],
    )
    def k(table_hbm, idx_hbm, out_hbm, idx_v, rows_v, sg0, sg1, ss0, ss1):
        wid = lax.axis_index("s") * _NC + lax.axis_index("c")
        row_base = _S_ROWS + wid * _ROWS_PER_TILE
        tok_base = row_base * _CHUNK
        pltpu.sync_copy(idx_hbm.at[pl.ds(row_base, _ROWS_PER_TILE)], idx_v)

        def gathers(g, half, sem, start):
            for j in range(_G):
                c = _G * g + j

                @pl.when(c < _ROWS_PER_TILE)
                def _():
                    cp = pltpu.make_async_copy(
                        table_hbm.at[idx_v.at[c]], rows_v.at[half * _G + j],
                        sem)
                    cp.start() if start else cp.wait()

        def scatters(g, half, sem, start):
            for j in range(_G):
                c = _G * g + j

                @pl.when(c < _ROWS_PER_TILE)
                def _():
                    cp = pltpu.make_async_copy(
                        rows_v.at[half * _G + j],
                        out_hbm.at[pl.ds(tok_base + c * _CHUNK, _CHUNK)], sem)
                    cp.start() if start else cp.wait()

        gathers(0, 0, sg0, True)

        def body(i, carry):
            ga = 2 * i
            gb = 2 * i + 1
            gathers(ga, 0, sg0, False)
            scatters(ga, 0, ss0, True)

            @pl.when(i >= 1)
            def _():
                scatters(ga - 1, 1, ss1, False)

            @pl.when(gb < _NGRP)
            def _():
                gathers(gb, 1, sg1, True)
                gathers(gb, 1, sg1, False)
                scatters(gb, 1, ss1, True)

            scatters(ga, 0, ss0, False)

            @pl.when(ga + 2 < _NGRP)
            def _():
                gathers(ga + 2, 0, sg0, True)

            return carry

        lax.fori_loop(0, (_NGRP + 1) // 2, body, 0)
        if _NGRP % 2 == 0:
            scatters(_NGRP - 1, 1, ss1, False)

    return k(table, idx2d)


def _bf16_body(tbl_ref, out_ref):
    out_ref[...] = tbl_ref[...].astype(jnp.bfloat16)


def _to_bf16(table):
    return pl.pallas_call(
        _bf16_body,
        out_shape=jax.ShapeDtypeStruct((_EXT, _D), jnp.bfloat16),
    )(table)


_TC_SUB = 8


def _tc_body(eff_ref, tbl_ref, out_ref):
    tbl = tbl_ref[...]
    v = lax.broadcasted_iota(jnp.int32, (_EXT, _CHUNK), 0)
    for r in range(_TC_SUB):
        oh = (v == eff_ref[r:r + 1, :]).astype(jnp.bfloat16)
        out_ref[r * _CHUNK:(r + 1) * _CHUNK, :] = lax.dot_general(
            oh, tbl, (((0,), (0,)), ((), ())),
            preferred_element_type=jnp.float32)


def _tc_gather(eff2d, tbl16):
    return pl.pallas_call(
        _tc_body,
        grid=(_S_ROWS // _TC_SUB,),
        in_specs=[
            pl.BlockSpec((_TC_SUB, _CHUNK), lambda i: (i, 0)),
            pl.BlockSpec((_EXT, _D), lambda i: (0, 0)),
        ],
        out_specs=pl.BlockSpec((_TC_SUB * _CHUNK, _D), lambda i: (i, 0)),
        out_shape=jax.ShapeDtypeStruct((_S_ROWS * _CHUNK, _D), jnp.float32),
    )(eff2d, tbl16)


def kernel(idx, token_weight, pitch_bias, pos_bias, bar_bias):
    idx = idx.astype(jnp.int32)
    table = _build_table(token_weight,
                         pitch_bias.reshape(1, _D),
                         pos_bias.reshape(1, _D),
                         bar_bias.reshape(1, _D))
    eff = _eff_idx(idx)
    idx2d = eff.reshape(_IDX_ROWS, _CHUNK)
    tbl16 = _to_bf16(table)
    sc_out = _sc_gather(table, idx2d)
    tc_out = _tc_gather(idx2d, tbl16)
    out = lax.dynamic_update_slice(sc_out, tc_out, (0, 0))
    return out.reshape(_B, _T, _D)


# speedup vs baseline: 3.1552x; 1.0054x over previous
import functools

import jax
import jax.numpy as jnp
from jax import lax
from jax.experimental import pallas as pl
from jax.experimental.pallas import tpu as pltpu
from jax.experimental.pallas import tpu_sc as plsc

_D = 128
_VOCAB = 1000
_BASE = 10000.0
_PITCH_SIZE = 128
_BAR_ID = 128
_POS_START, _POS_SIZE = 129, 32
_B, _T = 4096, 200
_EXT = 1208

_NC, _NS = 2, 16
_NW = _NC * _NS
_CHUNK = 128
_IDX_ROWS = _B * _T // _CHUNK
_S_ROWS = 1536
_ROWS_PER_TILE = (_IDX_ROWS - _S_ROWS) // _NW


def _table_body(tw_ref, pb_ref, qb_ref, bb_ref, out_ref):
    r = lax.broadcasted_iota(jnp.int32, (_EXT, _D), 0)
    d = lax.broadcasted_iota(jnp.int32, (_EXT, _D), 1)
    k = (d // 2).astype(jnp.float32)
    f = jnp.exp(k * jnp.float32(-2.0 / _D) * jnp.log(jnp.float32(_BASE)))
    is_pitch = r < _PITCH_SIZE
    is_pos = (r >= _POS_START) & (r < _POS_START + _POS_SIZE)
    n = jnp.where(is_pitch, r,
                  jnp.where(is_pos, r - _POS_START, r - _VOCAB)).astype(jnp.float32)
    ang = n * f
    even = (d % 2) == 0
    bias = jnp.where(is_pitch, pb_ref[...],
                     jnp.where(is_pos, qb_ref[...], bb_ref[...]))
    val = jnp.where(even, jnp.sin(ang), jnp.cos(ang)) + bias
    is_fme = is_pitch | is_pos
    out_ref[0:_VOCAB, :] = jnp.where(is_fme[0:_VOCAB, :], val[0:_VOCAB, :],
                                     tw_ref[...])
    out_ref[_VOCAB:_EXT, :] = val[_VOCAB:_EXT, :]


def _build_table(token_weight, pitch_bias, pos_bias, bar_bias):
    return pl.pallas_call(
        _table_body,
        out_shape=jax.ShapeDtypeStruct((_EXT, _D), jnp.float32),
    )(token_weight, pitch_bias, pos_bias, bar_bias)


def _eff_body(idx_ref, out_ref):
    x = idx_ref[...]
    bar = x == _BAR_ID
    barf = bar.astype(jnp.float32)
    ti = lax.broadcasted_iota(jnp.int32, (_T, _T), 0)
    tj = lax.broadcasted_iota(jnp.int32, (_T, _T), 1)
    tril = (ti <= tj).astype(jnp.float32)
    csum = lax.dot(barf, tril, preferred_element_type=jnp.float32)
    bar_val = jnp.maximum(csum.astype(jnp.int32) - 1, 0)
    out_ref[...] = jnp.where(bar, _VOCAB + bar_val, jnp.clip(x, 0, _VOCAB - 1))


def _eff_idx(idx):
    blk = 256
    return pl.pallas_call(
        _eff_body,
        grid=(_B // blk,),
        in_specs=[pl.BlockSpec((blk, _T), lambda i: (i, 0))],
        out_specs=pl.BlockSpec((blk, _T), lambda i: (i, 0)),
        out_shape=jax.ShapeDtypeStruct((_B, _T), jnp.int32),
    )(idx)


_G = 3
_NGRP = -(-_ROWS_PER_TILE // _G)


def _sc_gather(table, idx2d):
    mesh = plsc.VectorSubcoreMesh(core_axis_name="c", subcore_axis_name="s")

    @functools.partial(
        pl.kernel,
        out_type=jax.ShapeDtypeStruct((_B * _T, _D), jnp.float32),
        mesh=mesh,
        scratch_types=[
            pltpu.VMEM((_ROWS_PER_TILE, _CHUNK), jnp.int32),
            pltpu.VMEM((2 * _G, _CHUNK, _D), jnp.float32),
            pltpu.SemaphoreType.DMA,
            pltpu.SemaphoreType.DMA,
            pltpu.SemaphoreType.DMA,
            pltpu.SemaphoreType.DMA,
        ],
    )
    def k(table_hbm, idx_hbm, out_hbm, idx_v, rows_v, sg0, sg1, ss0, ss1):
        wid = lax.axis_index("s") * _NC + lax.axis_index("c")
        row_base = _S_ROWS + wid * _ROWS_PER_TILE
        tok_base = row_base * _CHUNK
        pltpu.sync_copy(idx_hbm.at[pl.ds(row_base, _ROWS_PER_TILE)], idx_v)

        def gathers(g, half, sem, start):
            for j in range(_G):
                c = _G * g + j

                @pl.when(c < _ROWS_PER_TILE)
                def _():
                    cp = pltpu.make_async_copy(
                        table_hbm.at[idx_v.at[c]], rows_v.at[half * _G + j],
                        sem)
                    cp.start() if start else cp.wait()

        def scatters(g, half, sem, start):
            for j in range(_G):
                c = _G * g + j

                @pl.when(c < _ROWS_PER_TILE)
                def _():
                    cp = pltpu.make_async_copy(
                        rows_v.at[half * _G + j],
                        out_hbm.at[pl.ds(tok_base + c * _CHUNK, _CHUNK)], sem)
                    cp.start() if start else cp.wait()

        gathers(0, 0, sg0, True)

        def body(i, carry):
            ga = 2 * i
            gb = 2 * i + 1
            gathers(ga, 0, sg0, False)
            scatters(ga, 0, ss0, True)

            @pl.when(i >= 1)
            def _():
                scatters(ga - 1, 1, ss1, False)

            @pl.when(gb < _NGRP)
            def _():
                gathers(gb, 1, sg1, True)
                gathers(gb, 1, sg1, False)
                scatters(gb, 1, ss1, True)

            scatters(ga, 0, ss0, False)

            @pl.when(ga + 2 < _NGRP)
            def _():
                gathers(ga + 2, 0, sg0, True)

            return carry

        lax.fori_loop(0, (_NGRP + 1) // 2, body, 0)
        if _NGRP % 2 == 0:
            scatters(_NGRP - 1, 1, ss1, False)

    return k(table, idx2d)


def _bf16_body(tbl_ref, out_ref):
    out_ref[...] = tbl_ref[...].astype(jnp.bfloat16)


def _to_bf16(table):
    return pl.pallas_call(
        _bf16_body,
        out_shape=jax.ShapeDtypeStruct((_EXT, _D), jnp.bfloat16),
    )(table)


_TC_SUB = 8


def _tc_body(eff_ref, tbl_ref, out_ref):
    tbl = tbl_ref[...]
    v = lax.broadcasted_iota(jnp.int32, (_EXT, _CHUNK), 0)
    for r in range(_TC_SUB):
        oh = (v == eff_ref[r:r + 1, :]).astype(jnp.bfloat16)
        out_ref[r * _CHUNK:(r + 1) * _CHUNK, :] = lax.dot_general(
            oh, tbl, (((0,), (0,)), ((), ())),
            preferred_element_type=jnp.float32)


def _tc_gather(eff2d, tbl16):
    return pl.pallas_call(
        _tc_body,
        grid=(_S_ROWS // _TC_SUB,),
        in_specs=[
            pl.BlockSpec((_TC_SUB, _CHUNK), lambda i: (i, 0)),
            pl.BlockSpec((_EXT, _D), lambda i: (0, 0)),
        ],
        out_specs=pl.BlockSpec((_TC_SUB * _CHUNK, _D), lambda i: (i, 0)),
        out_shape=jax.ShapeDtypeStruct((_S_ROWS * _CHUNK, _D), jnp.float32),
    )(eff2d, tbl16)


def kernel(idx, token_weight, pitch_bias, pos_bias, bar_bias):
    idx = idx.astype(jnp.int32)
    table = _build_table(token_weight,
                         pitch_bias.reshape(1, _D),
                         pos_bias.reshape(1, _D),
                         bar_bias.reshape(1, _D))
    eff = _eff_idx(idx)
    idx2d = eff.reshape(_IDX_ROWS, _CHUNK)
    tbl16 = _to_bf16(table)
    sc_out = _sc_gather(table, idx2d)
    tc_out = _tc_gather(idx2d, tbl16)
    out = lax.dynamic_update_slice(sc_out, tc_out, (0, 0))
    return out.reshape(_B, _T, _D)
